# Initial kernel scaffold; baseline (speedup 1.0000x reference)
import jax
import jax.numpy as jnp
from jax.experimental import pallas as pl


def kernel(h, edge_attr, W0, W1, W2, attn_w, weights, edge_index):
    raise NotImplementedError("write your pallas kernel here")


# speedup vs baseline: 6.7868x; 6.7868x over previous
import functools

import jax
import jax.numpy as jnp
from jax import lax
from jax.experimental import pallas as pl
from jax.experimental.pallas import tpu as pltpu
from jax.experimental.pallas import tpu_sc as plsc

_N = 10000
_NP = 10240
_E = 320000
_D = 128
_NW = 32
_EPT = 10240
_EP = _NW * _EPT
_CH = 128
_NCH = _EPT // _CH
_STR = _NP // 16

_f32 = jnp.float32
_i32 = jnp.int32


def _prep_body(h_ref, w1t_ref, w2t_ref, a1_ref, a2_ref,
               z_ref, zi_ref, s1_ref, s2_ref):
    hb = h_ref[...]
    z = jnp.dot(hb, w1t_ref[...], preferred_element_type=_f32)
    zi = jnp.dot(hb, w2t_ref[...], preferred_element_type=_f32)
    z_ref[...] = z
    zi_ref[...] = zi
    s1_ref[...] = jnp.dot(z, a1_ref[...], preferred_element_type=_f32)
    s2_ref[...] = jnp.dot(z, a2_ref[...], preferred_element_type=_f32)


def _tc_prep(h_p, w1t, w2t, a1, a2):
    nb = 8
    blk = _NP // nb
    return pl.pallas_call(
        _prep_body,
        grid=(nb,),
        in_specs=[
            pl.BlockSpec((blk, _D), lambda i: (i, 0)),
            pl.BlockSpec((_D, _D), lambda i: (0, 0)),
            pl.BlockSpec((_D, _D), lambda i: (0, 0)),
            pl.BlockSpec((_D, 1), lambda i: (0, 0)),
            pl.BlockSpec((_D, 1), lambda i: (0, 0)),
        ],
        out_specs=[
            pl.BlockSpec((blk, _D), lambda i: (i, 0)),
            pl.BlockSpec((blk, _D), lambda i: (i, 0)),
            pl.BlockSpec((blk, 1), lambda i: (i, 0)),
            pl.BlockSpec((blk, 1), lambda i: (i, 0)),
        ],
        out_shape=[
            jax.ShapeDtypeStruct((_NP, _D), _f32),
            jax.ShapeDtypeStruct((_NP, _D), _f32),
            jax.ShapeDtypeStruct((_NP, 1), _f32),
            jax.ShapeDtypeStruct((_NP, 1), _f32),
        ],
    )(h_p, w1t, w2t, a1, a2)


def _u_body(ea_ref, w0t_ref, a3_ref, u_ref):
    wu = jnp.dot(w0t_ref[...], a3_ref[...], preferred_element_type=_f32)
    u_ref[...] = jnp.dot(ea_ref[...], wu, preferred_element_type=_f32)


def _tc_u(edge_attr, w0t, a3):
    nb = 160
    blk = _E // nb
    return pl.pallas_call(
        _u_body,
        grid=(nb,),
        in_specs=[
            pl.BlockSpec((blk, 16), lambda i: (i, 0)),
            pl.BlockSpec((16, 16), lambda i: (0, 0)),
            pl.BlockSpec((16, 1), lambda i: (0, 0)),
        ],
        out_specs=pl.BlockSpec((blk, 1), lambda i: (i, 0)),
        out_shape=jax.ShapeDtypeStruct((_E, 1), _f32),
    )(edge_attr, w0t, a3)


def _sc1_body(src_hbm, dst_hbm, u_hbm, s1_hbm, s2_hbm,
              ex_hbm, dp_hbm,
              src_v, dst_v, u_v, ex_v, s1_v, s2_v, den_v):
    c = lax.axis_index("c")
    s = lax.axis_index("s")
    wid = c * 16 + s
    pltpu.sync_copy(src_hbm.at[wid], src_v)
    pltpu.sync_copy(dst_hbm.at[wid], dst_v)
    pltpu.sync_copy(u_hbm.at[wid], u_v)
    pltpu.sync_copy(s1_hbm, s1_v)
    pltpu.sync_copy(s2_hbm, s2_v)

    zero16 = jnp.zeros((16,), _f32)

    def zero_body(i, carry):
        den_v[pl.ds(i * 16, 16)] = zero16
        return carry

    lax.fori_loop(0, _NP // 16, zero_body, 0)

    def edge_body(i, carry):
        sl = src_v[pl.ds(i * 16, 16)]
        dl = dst_v[pl.ds(i * 16, 16)]
        g1 = plsc.load_gather(s1_v, [sl])
        g2 = plsc.load_gather(s2_v, [dl])
        x = g1 + g2 + u_v[pl.ds(i * 16, 16)]
        x = jnp.maximum(x, 0.01 * x)
        ex = jnp.exp(x)
        ex_v[pl.ds(i * 16, 16)] = ex
        plsc.addupdate_scatter(den_v, [dl], ex)
        return carry

    lax.fori_loop(0, _EPT // 16, edge_body, 0)

    pltpu.sync_copy(ex_v, ex_hbm.at[wid])
    pltpu.sync_copy(den_v, dp_hbm.at[wid])


def _sc_pass1(src_p, dst_p, u_p, s1, s2):
    mesh = plsc.VectorSubcoreMesh(core_axis_name="c", subcore_axis_name="s")
    return pl.kernel(
        _sc1_body,
        out_type=[
            jax.ShapeDtypeStruct((_NW, _EPT), _f32),
            jax.ShapeDtypeStruct((_NW, _NP), _f32),
        ],
        mesh=mesh,
        scratch_types=[
            pltpu.VMEM((_EPT,), _i32),
            pltpu.VMEM((_EPT,), _i32),
            pltpu.VMEM((_EPT,), _f32),
            pltpu.VMEM((_EPT,), _f32),
            pltpu.VMEM((_NP,), _f32),
            pltpu.VMEM((_NP,), _f32),
            pltpu.VMEM((_NP,), _f32),
        ],
        compiler_params=pltpu.CompilerParams(needs_layout_passes=False),
    )(src_p, dst_p, u_p, s1, s2)


def _sc2_body(src_hbm, dst3_hbm, ex_hbm, dp_hbm, z_hbm,
              out0, out1,
              src_c, dst_v, ex_c, den_v, tmp_v, stripe_v, alpha_v, rows_v,
              zn_sh, sd_sh, sem):
    c = lax.axis_index("c")
    s = lax.axis_index("s")
    wid = c * 16 + s
    base = s * _STR
    zero16 = jnp.zeros((16,), _f32)

    pltpu.sync_copy(dst3_hbm.at[wid], dst_v)

    def zstripe(i, carry):
        stripe_v[pl.ds(i * 16, 16)] = zero16
        return carry

    lax.fori_loop(0, _STR // 16, zstripe, 0)

    def red_outer(p, carry):
        pltpu.sync_copy(dp_hbm.at[p, pl.ds(base, _STR)], tmp_v)

        def red_inner(i, c2):
            off = pl.ds(i * 16, 16)
            stripe_v[off] = stripe_v[off] + tmp_v[off]
            return c2

        lax.fori_loop(0, _STR // 16, red_inner, 0)
        return carry

    lax.fori_loop(0, _NW, red_outer, 0)
    pltpu.sync_copy(stripe_v, sd_sh.at[pl.ds(base, _STR)])

    def zrows(j, carry):
        for q in range(8):
            rows_v[j, pl.ds(q * 16, 16)] = zero16
        return carry

    lax.fori_loop(0, _CH, zrows, 0)
    for i in range(_STR // _CH):
        pltpu.sync_copy(rows_v, zn_sh.at[pl.ds(base + i * _CH, _CH)])

    plsc.subcore_barrier()
    pltpu.sync_copy(sd_sh, den_v)

    def chunk_body(j, carry):
        pltpu.sync_copy(src_hbm.at[wid, pl.ds(j * _CH, _CH)], src_c)
        pltpu.sync_copy(ex_hbm.at[wid, pl.ds(j * _CH, _CH)], ex_c)
        cp = pltpu.async_copy(z_hbm.at[src_c], rows_v, sem)
        for k in range(8):
            dl = dst_v[j, pl.ds(k * 16, 16)]
            d = plsc.load_gather(den_v, [dl])
            exv = ex_c[pl.ds(k * 16, 16)]
            al = jnp.where(d > 0.0, exv / d, zero16)
            alpha_v[pl.ds(k * 16, 16)] = al
        cp.wait()

        def scale_body(g, c2):
            a16 = alpha_v[pl.ds(g * 16, 16)]
            for rr in range(16):
                av = jnp.full((16,), a16[rr], _f32)
                r = g * 16 + rr
                for q in range(8):
                    off = pl.ds(q * 16, 16)
                    rows_v[r, off] = rows_v[r, off] * av
            return c2

        lax.fori_loop(0, _CH // 16, scale_body, 0)
        pltpu.sync_copy(rows_v, zn_sh.at[dst_v.at[j]], add=True)
        return carry

    lax.fori_loop(0, _NCH, chunk_body, 0)

    plsc.subcore_barrier()

    @pl.when(c == 0)
    def _():
        pltpu.sync_copy(zn_sh.at[pl.ds(base, _STR)],
                        out0.at[pl.ds(base, _STR)])

    @pl.when(c == 1)
    def _():
        pltpu.sync_copy(zn_sh.at[pl.ds(base, _STR)],
                        out1.at[pl.ds(base, _STR)])


def _sc_pass2(src_p, dst3, ex, dp, z):
    mesh = plsc.VectorSubcoreMesh(core_axis_name="c", subcore_axis_name="s")
    return pl.kernel(
        _sc2_body,
        out_type=[
            jax.ShapeDtypeStruct((_NP, _D), _f32),
            jax.ShapeDtypeStruct((_NP, _D), _f32),
        ],
        mesh=mesh,
        scratch_types=[
            pltpu.VMEM((_CH,), _i32),
            pltpu.VMEM((_NCH, _CH), _i32),
            pltpu.VMEM((_CH,), _f32),
            pltpu.VMEM((_NP,), _f32),
            pltpu.VMEM((_STR,), _f32),
            pltpu.VMEM((_STR,), _f32),
            pltpu.VMEM((_CH,), _f32),
            pltpu.VMEM((_CH, _D), _f32),
            pltpu.VMEM_SHARED((_NP, _D), _f32),
            pltpu.VMEM_SHARED((_NP,), _f32),
            pltpu.SemaphoreType.DMA,
        ],
        compiler_params=pltpu.CompilerParams(needs_layout_passes=False),
    )(src_p, dst3, ex, dp, z)


def _epi_body(zi_ref, zn0_ref, zn1_ref, out_ref):
    out_ref[...] = jnp.maximum(
        zi_ref[...] + zn0_ref[...] + zn1_ref[...], 0.0)


def _tc_epi(zi, zn0, zn1):
    nb = 8
    blk = _NP // nb
    return pl.pallas_call(
        _epi_body,
        grid=(nb,),
        in_specs=[
            pl.BlockSpec((blk, _D), lambda i: (i, 0)),
            pl.BlockSpec((blk, _D), lambda i: (i, 0)),
            pl.BlockSpec((blk, _D), lambda i: (i, 0)),
        ],
        out_specs=pl.BlockSpec((blk, _D), lambda i: (i, 0)),
        out_shape=jax.ShapeDtypeStruct((_NP, _D), _f32),
    )(zi, zn0, zn1)


@jax.jit
def kernel(h, edge_attr, W0, W1, W2, attn_w, weights, edge_index):
    del weights
    src = edge_index[0]
    dst = edge_index[1]

    a1 = attn_w[0, :_D].reshape(_D, 1)
    a2 = attn_w[0, _D:2 * _D].reshape(_D, 1)
    a3 = attn_w[0, 2 * _D:].reshape(16, 1)

    h_p = jnp.pad(h, ((0, _NP - _N), (0, 0)))
    z, zi, s1, s2 = _tc_prep(h_p, W1.T, W2.T, a1, a2)
    u = _tc_u(edge_attr, W0.T, a3)

    pad_e = _EP - _E
    src_p = jnp.concatenate(
        [src, jnp.zeros((pad_e,), _i32)]).reshape(_NW, _EPT)
    dst_p = jnp.concatenate(
        [dst, jnp.zeros((pad_e,), _i32)]).reshape(_NW, _EPT)
    dst3 = dst_p.reshape(_NW, _NCH, _CH)
    u_p = jnp.concatenate(
        [u[:, 0], jnp.full((pad_e,), -jnp.inf, _f32)]).reshape(_NW, _EPT)

    ex, dp = _sc_pass1(src_p, dst_p, u_p, s1[:, 0], s2[:, 0])
    zn0, zn1 = _sc_pass2(src_p, dst3, ex, dp, z)

    out = _tc_epi(zi, zn0, zn1)
    return out[:_N]


# speedup vs baseline: 8.7222x; 1.2852x over previous
import functools

import jax
import jax.numpy as jnp
from jax import lax
from jax.experimental import pallas as pl
from jax.experimental.pallas import tpu as pltpu
from jax.experimental.pallas import tpu_sc as plsc

_N = 10000
_NP = 10240
_E = 320000
_D = 128
_NW = 32
_EPT = 10240
_EP = _NW * _EPT
_CH = 128
_NCH = _EPT // _CH
_STR = _NP // 16

_f32 = jnp.float32
_i32 = jnp.int32


def _prep_body(h_ref, w1t_ref, w2t_ref, a1_ref, a2_ref,
               z_ref, zi_ref, s1_ref, s2_ref):
    hb = h_ref[...]
    z = jnp.dot(hb, w1t_ref[...], preferred_element_type=_f32)
    zi = jnp.dot(hb, w2t_ref[...], preferred_element_type=_f32)
    z_ref[...] = z
    zi_ref[...] = zi
    s1_ref[...] = jnp.dot(z, a1_ref[...], preferred_element_type=_f32)
    s2_ref[...] = jnp.dot(z, a2_ref[...], preferred_element_type=_f32)


def _tc_prep(h_p, w1t, w2t, a1, a2):
    nb = 8
    blk = _NP // nb
    return pl.pallas_call(
        _prep_body,
        grid=(nb,),
        in_specs=[
            pl.BlockSpec((blk, _D), lambda i: (i, 0)),
            pl.BlockSpec((_D, _D), lambda i: (0, 0)),
            pl.BlockSpec((_D, _D), lambda i: (0, 0)),
            pl.BlockSpec((_D, 1), lambda i: (0, 0)),
            pl.BlockSpec((_D, 1), lambda i: (0, 0)),
        ],
        out_specs=[
            pl.BlockSpec((blk, _D), lambda i: (i, 0)),
            pl.BlockSpec((blk, _D), lambda i: (i, 0)),
            pl.BlockSpec((blk, 1), lambda i: (i, 0)),
            pl.BlockSpec((blk, 1), lambda i: (i, 0)),
        ],
        out_shape=[
            jax.ShapeDtypeStruct((_NP, _D), _f32),
            jax.ShapeDtypeStruct((_NP, _D), _f32),
            jax.ShapeDtypeStruct((_NP, 1), _f32),
            jax.ShapeDtypeStruct((_NP, 1), _f32),
        ],
    )(h_p, w1t, w2t, a1, a2)


def _u_body(ea_ref, w0t_ref, a3_ref, u_ref):
    wu = jnp.dot(w0t_ref[...], a3_ref[...], preferred_element_type=_f32)
    u_ref[...] = jnp.dot(ea_ref[...], wu, preferred_element_type=_f32)


def _tc_u(edge_attr, w0t, a3):
    nb = 160
    blk = _E // nb
    return pl.pallas_call(
        _u_body,
        grid=(nb,),
        in_specs=[
            pl.BlockSpec((blk, 16), lambda i: (i, 0)),
            pl.BlockSpec((16, 16), lambda i: (0, 0)),
            pl.BlockSpec((16, 1), lambda i: (0, 0)),
        ],
        out_specs=pl.BlockSpec((blk, 1), lambda i: (i, 0)),
        out_shape=jax.ShapeDtypeStruct((_E, 1), _f32),
    )(edge_attr, w0t, a3)


def _sc1_body(src_hbm, dst_hbm, u_hbm, s1_hbm, s2_hbm,
              ex_hbm, dp_hbm,
              src_v, dst_v, u_v, ex_v, s1_v, s2_v, den_v):
    c = lax.axis_index("c")
    s = lax.axis_index("s")
    wid = c * 16 + s
    pltpu.sync_copy(src_hbm.at[wid], src_v)
    pltpu.sync_copy(dst_hbm.at[wid], dst_v)
    pltpu.sync_copy(u_hbm.at[wid], u_v)
    pltpu.sync_copy(s1_hbm, s1_v)
    pltpu.sync_copy(s2_hbm, s2_v)

    zero16 = jnp.zeros((16,), _f32)

    def zero_body(i, carry):
        den_v[pl.ds(i * 16, 16)] = zero16
        return carry

    lax.fori_loop(0, _NP // 16, zero_body, 0)

    def edge_body(i, carry):
        sl = src_v[pl.ds(i * 16, 16)]
        dl = dst_v[pl.ds(i * 16, 16)]
        g1 = plsc.load_gather(s1_v, [sl])
        g2 = plsc.load_gather(s2_v, [dl])
        x = g1 + g2 + u_v[pl.ds(i * 16, 16)]
        x = jnp.maximum(x, 0.01 * x)
        ex = jnp.exp(x)
        ex_v[pl.ds(i * 16, 16)] = ex
        plsc.addupdate_scatter(den_v, [dl], ex)
        return carry

    lax.fori_loop(0, _EPT // 16, edge_body, 0)

    pltpu.sync_copy(ex_v, ex_hbm.at[wid])
    pltpu.sync_copy(den_v, dp_hbm.at[wid])


def _sc_pass1(src_p, dst_p, u_p, s1, s2):
    mesh = plsc.VectorSubcoreMesh(core_axis_name="c", subcore_axis_name="s")
    return pl.kernel(
        _sc1_body,
        out_type=[
            jax.ShapeDtypeStruct((_NW, _EPT), _f32),
            jax.ShapeDtypeStruct((_NW, _NP), _f32),
        ],
        mesh=mesh,
        scratch_types=[
            pltpu.VMEM((_EPT,), _i32),
            pltpu.VMEM((_EPT,), _i32),
            pltpu.VMEM((_EPT,), _f32),
            pltpu.VMEM((_EPT,), _f32),
            pltpu.VMEM((_NP,), _f32),
            pltpu.VMEM((_NP,), _f32),
            pltpu.VMEM((_NP,), _f32),
        ],
        compiler_params=pltpu.CompilerParams(needs_layout_passes=False),
    )(src_p, dst_p, u_p, s1, s2)


def _sc2_body(sd_hbm, ex_hbm, dp_hbm, z_hbm,
              out0, out1,
              sd_c0, sd_c1, ex_c0, ex_c1, dst_s0, dst_s1,
              rows0, rows1, den_v, tmp_v, stripe_v, alpha_v,
              zn_sh, sd_sh,
              sem_i0, sem_i1, sem_g0, sem_g1, sem_s0, sem_s1):
    c = lax.axis_index("c")
    s = lax.axis_index("s")
    wid = c * 16 + s
    base = s * _STR
    zero16 = jnp.zeros((16,), _f32)

    sd_c = (sd_c0, sd_c1)
    ex_c = (ex_c0, ex_c1)
    dst_s = (dst_s0, dst_s1)
    rows = (rows0, rows1)
    sem_i = (sem_i0, sem_i1)
    sem_g = (sem_g0, sem_g1)
    sem_s = (sem_s0, sem_s1)

    def zstripe(i, carry):
        stripe_v[pl.ds(i * 16, 16)] = zero16
        return carry

    lax.fori_loop(0, _STR // 16, zstripe, 0)

    def red_outer(p, carry):
        pltpu.sync_copy(dp_hbm.at[p, pl.ds(base, _STR)], tmp_v)

        def red_inner(i, c2):
            off = pl.ds(i * 16, 16)
            stripe_v[off] = stripe_v[off] + tmp_v[off]
            return c2

        lax.fori_loop(0, _STR // 16, red_inner, 0)
        return carry

    lax.fori_loop(0, _NW, red_outer, 0)
    pltpu.sync_copy(stripe_v, sd_sh.at[pl.ds(base, _STR)])

    def zrows(j, carry):
        for q in range(8):
            rows0[j, pl.ds(q * 16, 16)] = zero16
        return carry

    lax.fori_loop(0, _CH, zrows, 0)
    for i in range(_STR // _CH):
        pltpu.sync_copy(rows0, zn_sh.at[pl.ds(base + i * _CH, _CH)])

    pltpu.async_copy(sd_hbm.at[wid, 0], sd_c0, sem_i0)
    pltpu.async_copy(ex_hbm.at[wid, pl.ds(0, _CH)], ex_c0, sem_i0)
    pltpu.async_copy(sd_hbm.at[wid, 1], sd_c1, sem_i1)
    pltpu.async_copy(ex_hbm.at[wid, pl.ds(_CH, _CH)], ex_c1, sem_i1)
    pltpu.make_async_copy(sd_hbm.at[wid, 0], sd_c0, sem_i0).wait()
    pltpu.make_async_copy(ex_hbm.at[wid, pl.ds(0, _CH)], ex_c0, sem_i0).wait()
    pltpu.async_copy(z_hbm.at[sd_c0.at[0]], rows0, sem_g0)

    plsc.subcore_barrier()
    pltpu.sync_copy(sd_sh, den_v)

    def pair_body(g, carry):
        for b in range(2):
            ob = 1 - b
            j = 2 * g + b

            @pl.when(j >= 1)
            def _():
                pltpu.make_async_copy(
                    rows[ob], zn_sh.at[dst_s[ob].at[0]], sem_s[ob]).wait()

            @pl.when(j + 1 < _NCH)
            def _():
                pltpu.make_async_copy(
                    sd_hbm.at[wid, 0], sd_c[ob], sem_i[ob]).wait()
                pltpu.make_async_copy(
                    ex_hbm.at[wid, pl.ds(0, _CH)], ex_c[ob],
                    sem_i[ob]).wait()
                pltpu.async_copy(
                    z_hbm.at[sd_c[ob].at[0]], rows[ob], sem_g[ob])

            pltpu.make_async_copy(
                z_hbm.at[sd_c[b].at[0]], rows[b], sem_g[b]).wait()

            for k in range(8):
                off = pl.ds(k * 16, 16)
                dl = sd_c[b][1, off]
                d = plsc.load_gather(den_v, [dl])
                exv = ex_c[b][off]
                al = jnp.where(d > 0.0, exv / d, zero16)
                alpha_v[off] = al
                dst_s[b][0, off] = dl

            def scale_body(gg, c2):
                a16 = alpha_v[pl.ds(gg * 16, 16)]
                for rr in range(16):
                    av = jnp.full((16,), a16[rr], _f32)
                    r = gg * 16 + rr
                    for q in range(8):
                        off = pl.ds(q * 16, 16)
                        rows[b][r, off] = rows[b][r, off] * av
                return c2

            lax.fori_loop(0, _CH // 16, scale_body, 0)

            pltpu.async_copy(
                rows[b], zn_sh.at[dst_s[b].at[0]], sem_s[b], add=True)

            @pl.when(j + 2 < _NCH)
            def _():
                pltpu.async_copy(
                    sd_hbm.at[wid, j + 2], sd_c[b], sem_i[b])
                pltpu.async_copy(
                    ex_hbm.at[wid, pl.ds((j + 2) * _CH, _CH)], ex_c[b],
                    sem_i[b])
        return carry

    lax.fori_loop(0, _NCH // 2, pair_body, 0)

    pltpu.make_async_copy(rows1, zn_sh.at[dst_s1.at[0]], sem_s1).wait()

    plsc.subcore_barrier()

    @pl.when(c == 0)
    def _():
        pltpu.sync_copy(zn_sh.at[pl.ds(base, _STR)],
                        out0.at[pl.ds(base, _STR)])

    @pl.when(c == 1)
    def _():
        pltpu.sync_copy(zn_sh.at[pl.ds(base, _STR)],
                        out1.at[pl.ds(base, _STR)])


def _sc_pass2(sd, ex, dp, z):
    mesh = plsc.VectorSubcoreMesh(core_axis_name="c", subcore_axis_name="s")
    return pl.kernel(
        _sc2_body,
        out_type=[
            jax.ShapeDtypeStruct((_NP, _D), _f32),
            jax.ShapeDtypeStruct((_NP, _D), _f32),
        ],
        mesh=mesh,
        scratch_types=[
            pltpu.VMEM((2, _CH), _i32),
            pltpu.VMEM((2, _CH), _i32),
            pltpu.VMEM((_CH,), _f32),
            pltpu.VMEM((_CH,), _f32),
            pltpu.VMEM((1, _CH), _i32),
            pltpu.VMEM((1, _CH), _i32),
            pltpu.VMEM((_CH, _D), _f32),
            pltpu.VMEM((_CH, _D), _f32),
            pltpu.VMEM((_NP,), _f32),
            pltpu.VMEM((_STR,), _f32),
            pltpu.VMEM((_STR,), _f32),
            pltpu.VMEM((_CH,), _f32),
            pltpu.VMEM_SHARED((_NP, _D), _f32),
            pltpu.VMEM_SHARED((_NP,), _f32),
            pltpu.SemaphoreType.DMA,
            pltpu.SemaphoreType.DMA,
            pltpu.SemaphoreType.DMA,
            pltpu.SemaphoreType.DMA,
            pltpu.SemaphoreType.DMA,
            pltpu.SemaphoreType.DMA,
        ],
        compiler_params=pltpu.CompilerParams(needs_layout_passes=False),
    )(sd, ex, dp, z)


def _epi_body(zi_ref, zn0_ref, zn1_ref, out_ref):
    out_ref[...] = jnp.maximum(
        zi_ref[...] + zn0_ref[...] + zn1_ref[...], 0.0)


def _tc_epi(zi, zn0, zn1):
    nb = 8
    blk = _NP // nb
    return pl.pallas_call(
        _epi_body,
        grid=(nb,),
        in_specs=[
            pl.BlockSpec((blk, _D), lambda i: (i, 0)),
            pl.BlockSpec((blk, _D), lambda i: (i, 0)),
            pl.BlockSpec((blk, _D), lambda i: (i, 0)),
        ],
        out_specs=pl.BlockSpec((blk, _D), lambda i: (i, 0)),
        out_shape=jax.ShapeDtypeStruct((_NP, _D), _f32),
    )(zi, zn0, zn1)


@jax.jit
def kernel(h, edge_attr, W0, W1, W2, attn_w, weights, edge_index):
    del weights
    src = edge_index[0]
    dst = edge_index[1]

    a1 = attn_w[0, :_D].reshape(_D, 1)
    a2 = attn_w[0, _D:2 * _D].reshape(_D, 1)
    a3 = attn_w[0, 2 * _D:].reshape(16, 1)

    h_p = jnp.pad(h, ((0, _NP - _N), (0, 0)))
    z, zi, s1, s2 = _tc_prep(h_p, W1.T, W2.T, a1, a2)
    u = _tc_u(edge_attr, W0.T, a3)

    pad_e = _EP - _E
    src_p = jnp.concatenate(
        [src, jnp.zeros((pad_e,), _i32)]).reshape(_NW, _EPT)
    dst_p = jnp.concatenate(
        [dst, jnp.zeros((pad_e,), _i32)]).reshape(_NW, _EPT)
    sd = jnp.stack([src_p.reshape(_NW, _NCH, _CH),
                    dst_p.reshape(_NW, _NCH, _CH)], axis=2)
    u_p = jnp.concatenate(
        [u[:, 0], jnp.full((pad_e,), -jnp.inf, _f32)]).reshape(_NW, _EPT)

    ex, dp = _sc_pass1(src_p, dst_p, u_p, s1[:, 0], s2[:, 0])
    zn0, zn1 = _sc_pass2(sd, ex, dp, z)

    out = _tc_epi(zi, zn0, zn1)
    return out[:_N]


# speedup vs baseline: 10.9696x; 1.2577x over previous
import functools

import jax
import jax.numpy as jnp
from jax import lax
from jax.experimental import pallas as pl
from jax.experimental.pallas import tpu as pltpu
from jax.experimental.pallas import tpu_sc as plsc

_N = 10000
_NP = 10240
_E = 320000
_D = 128
_NW = 32
_EPT = 10240
_EP = _NW * _EPT
_CH = 128
_NCH = _EPT // _CH
_STR = _NP // 16

_f32 = jnp.float32
_i32 = jnp.int32


def _prep_body(h_ref, w1t_ref, w2t_ref, a1_ref, a2_ref,
               z_ref, zi_ref, s1_ref, s2_ref):
    hb = h_ref[...]
    z = jnp.dot(hb, w1t_ref[...], preferred_element_type=_f32)
    zi = jnp.dot(hb, w2t_ref[...], preferred_element_type=_f32)
    z_ref[...] = z
    zi_ref[...] = zi
    s1_ref[...] = jnp.dot(z, a1_ref[...], preferred_element_type=_f32)
    s2_ref[...] = jnp.dot(z, a2_ref[...], preferred_element_type=_f32)


def _tc_prep(h_p, w1t, w2t, a1, a2):
    nb = 8
    blk = _NP // nb
    return pl.pallas_call(
        _prep_body,
        grid=(nb,),
        in_specs=[
            pl.BlockSpec((blk, _D), lambda i: (i, 0)),
            pl.BlockSpec((_D, _D), lambda i: (0, 0)),
            pl.BlockSpec((_D, _D), lambda i: (0, 0)),
            pl.BlockSpec((_D, 1), lambda i: (0, 0)),
            pl.BlockSpec((_D, 1), lambda i: (0, 0)),
        ],
        out_specs=[
            pl.BlockSpec((blk, _D), lambda i: (i, 0)),
            pl.BlockSpec((blk, _D), lambda i: (i, 0)),
            pl.BlockSpec((blk, 1), lambda i: (i, 0)),
            pl.BlockSpec((blk, 1), lambda i: (i, 0)),
        ],
        out_shape=[
            jax.ShapeDtypeStruct((_NP, _D), _f32),
            jax.ShapeDtypeStruct((_NP, _D), _f32),
            jax.ShapeDtypeStruct((_NP, 1), _f32),
            jax.ShapeDtypeStruct((_NP, 1), _f32),
        ],
    )(h_p, w1t, w2t, a1, a2)


def _u_body(ea_ref, w0t_ref, a3_ref, u_ref):
    wu = jnp.dot(w0t_ref[...], a3_ref[...], preferred_element_type=_f32)
    r = lax.broadcasted_iota(_i32, (_D, 8), 0)
    cc = lax.broadcasted_iota(_i32, (_D, 8), 1)
    tiled = jnp.broadcast_to(wu.reshape(1, 16, 1), (8, 16, 8)).reshape(_D, 8)
    bd = jnp.where(r // 16 == cc, tiled, 0.0)
    u_ref[...] = jnp.dot(ea_ref[...], bd, preferred_element_type=_f32)


def _tc_u(ea2, w0t, a3):
    nb = 20
    blk = (_E // 8) // nb
    return pl.pallas_call(
        _u_body,
        grid=(nb,),
        in_specs=[
            pl.BlockSpec((blk, _D), lambda i: (i, 0)),
            pl.BlockSpec((16, 16), lambda i: (0, 0)),
            pl.BlockSpec((16, 1), lambda i: (0, 0)),
        ],
        out_specs=pl.BlockSpec((blk, 8), lambda i: (i, 0)),
        out_shape=jax.ShapeDtypeStruct((_E // 8, 8), _f32),
    )(ea2, w0t, a3)


def _sc1_body(src_hbm, dst_hbm, u_hbm, s1_hbm, s2_hbm,
              ex_hbm, dp_hbm,
              src_v, dst_v, u_v, ex_v, s1_v, s2_v, den_v):
    c = lax.axis_index("c")
    s = lax.axis_index("s")
    wid = c * 16 + s
    pltpu.sync_copy(src_hbm.at[wid], src_v)
    pltpu.sync_copy(dst_hbm.at[wid], dst_v)
    pltpu.sync_copy(u_hbm.at[wid], u_v)
    pltpu.sync_copy(s1_hbm, s1_v)
    pltpu.sync_copy(s2_hbm, s2_v)

    zero16 = jnp.zeros((16,), _f32)

    def zero_body(i, carry):
        den_v[pl.ds(i * 16, 16)] = zero16
        return carry

    lax.fori_loop(0, _NP // 16, zero_body, 0)

    def edge_body(i, carry):
        sl = src_v[pl.ds(i * 16, 16)]
        dl = dst_v[pl.ds(i * 16, 16)]
        g1 = plsc.load_gather(s1_v, [sl])
        g2 = plsc.load_gather(s2_v, [dl])
        x = g1 + g2 + u_v[pl.ds(i * 16, 16)]
        x = jnp.maximum(x, 0.01 * x)
        ex = jnp.exp(x)
        ex_v[pl.ds(i * 16, 16)] = ex
        plsc.addupdate_scatter(den_v, [dl], ex)
        return carry

    lax.fori_loop(0, _EPT // 16, edge_body, 0)

    pltpu.sync_copy(ex_v, ex_hbm.at[wid])
    pltpu.sync_copy(den_v, dp_hbm.at[wid])


def _sc_pass1(src_p, dst_p, u_p, s1, s2):
    mesh = plsc.VectorSubcoreMesh(core_axis_name="c", subcore_axis_name="s")
    return pl.kernel(
        _sc1_body,
        out_type=[
            jax.ShapeDtypeStruct((_NW, _EPT), _f32),
            jax.ShapeDtypeStruct((_NW, _NP), _f32),
        ],
        mesh=mesh,
        scratch_types=[
            pltpu.VMEM((_EPT,), _i32),
            pltpu.VMEM((_EPT,), _i32),
            pltpu.VMEM((_EPT,), _f32),
            pltpu.VMEM((_EPT,), _f32),
            pltpu.VMEM((_NP,), _f32),
            pltpu.VMEM((_NP,), _f32),
            pltpu.VMEM((_NP,), _f32),
        ],
        compiler_params=pltpu.CompilerParams(needs_layout_passes=False),
    )(src_p, dst_p, u_p, s1, s2)


def _sc2_body(src4_hbm, dst4_hbm, ex_hbm, dp_hbm, z_hbm,
              out0, out1,
              src_c0, src_c1, dst_c0, dst_c1, ex_c0, ex_c1, dst_s0, dst_s1,
              rows0, rows1, den_v, tmp_v, stripe_v, alpha_v,
              zn_sh, sd_sh,
              sem_i0, sem_i1, sem_g0, sem_g1, sem_s0, sem_s1):
    c = lax.axis_index("c")
    s = lax.axis_index("s")
    wid = c * 16 + s
    base = s * _STR
    zero16 = jnp.zeros((16,), _f32)

    src_c = (src_c0, src_c1)
    dst_c = (dst_c0, dst_c1)
    ex_c = (ex_c0, ex_c1)
    dst_s = (dst_s0, dst_s1)
    rows = (rows0, rows1)
    sem_i = (sem_i0, sem_i1)
    sem_g = (sem_g0, sem_g1)
    sem_s = (sem_s0, sem_s1)

    def zstripe(i, carry):
        stripe_v[pl.ds(i * 16, 16)] = zero16
        return carry

    lax.fori_loop(0, _STR // 16, zstripe, 0)

    def red_outer(p, carry):
        pltpu.sync_copy(dp_hbm.at[p, pl.ds(base, _STR)], tmp_v)

        def red_inner(i, c2):
            off = pl.ds(i * 16, 16)
            stripe_v[off] = stripe_v[off] + tmp_v[off]
            return c2

        lax.fori_loop(0, _STR // 16, red_inner, 0)
        return carry

    lax.fori_loop(0, _NW, red_outer, 0)
    pltpu.sync_copy(stripe_v, sd_sh.at[pl.ds(base, _STR)])

    def zrows(j, carry):
        for q in range(8):
            rows0[j, pl.ds(q * 16, 16)] = zero16
        return carry

    lax.fori_loop(0, _CH, zrows, 0)
    for i in range(_STR // _CH):
        pltpu.sync_copy(rows0, zn_sh.at[pl.ds(base + i * _CH, _CH)])

    pltpu.async_copy(src4_hbm.at[wid, 0], src_c0, sem_i0)
    pltpu.async_copy(dst4_hbm.at[wid, 0], dst_c0, sem_i0)
    pltpu.async_copy(ex_hbm.at[wid, pl.ds(0, _CH)], ex_c0, sem_i0)
    pltpu.async_copy(src4_hbm.at[wid, 1], src_c1, sem_i1)
    pltpu.async_copy(dst4_hbm.at[wid, 1], dst_c1, sem_i1)
    pltpu.async_copy(ex_hbm.at[wid, pl.ds(_CH, _CH)], ex_c1, sem_i1)
    pltpu.make_async_copy(src4_hbm.at[wid, 0], src_c0, sem_i0).wait()
    pltpu.make_async_copy(dst4_hbm.at[wid, 0], dst_c0, sem_i0).wait()
    pltpu.make_async_copy(ex_hbm.at[wid, pl.ds(0, _CH)], ex_c0, sem_i0).wait()
    pltpu.async_copy(z_hbm.at[src_c0.at[0]], rows0, sem_g0)

    plsc.subcore_barrier()
    pltpu.sync_copy(sd_sh, den_v)

    def pair_body(g, carry):
        for b in range(2):
            ob = 1 - b
            j = 2 * g + b

            @pl.when(j >= 1)
            def _():
                pltpu.make_async_copy(
                    rows[ob], zn_sh.at[dst_s[ob].at[0]], sem_s[ob]).wait()

            @pl.when(j + 1 < _NCH)
            def _():
                pltpu.make_async_copy(
                    src4_hbm.at[wid, 0], src_c[ob], sem_i[ob]).wait()
                pltpu.make_async_copy(
                    dst4_hbm.at[wid, 0], dst_c[ob], sem_i[ob]).wait()
                pltpu.make_async_copy(
                    ex_hbm.at[wid, pl.ds(0, _CH)], ex_c[ob],
                    sem_i[ob]).wait()
                pltpu.async_copy(
                    z_hbm.at[src_c[ob].at[0]], rows[ob], sem_g[ob])

            pltpu.make_async_copy(
                z_hbm.at[src_c[b].at[0]], rows[b], sem_g[b]).wait()

            for k in range(8):
                off = pl.ds(k * 16, 16)
                dl = dst_c[b][0, off]
                d = plsc.load_gather(den_v, [dl])
                exv = ex_c[b][off]
                al = jnp.where(d > 0.0, exv / d, zero16)
                alpha_v[off] = al
                dst_s[b][0, off] = dl

            def scale_body(gg, c2):
                a16 = alpha_v[pl.ds(gg * 16, 16)]
                for rr in range(16):
                    av = jnp.full((16,), a16[rr], _f32)
                    r = gg * 16 + rr
                    for q in range(8):
                        off = pl.ds(q * 16, 16)
                        rows[b][r, off] = rows[b][r, off] * av
                return c2

            lax.fori_loop(0, _CH // 16, scale_body, 0)

            pltpu.async_copy(
                rows[b], zn_sh.at[dst_s[b].at[0]], sem_s[b], add=True)

            @pl.when(j + 2 < _NCH)
            def _():
                pltpu.async_copy(
                    src4_hbm.at[wid, j + 2], src_c[b], sem_i[b])
                pltpu.async_copy(
                    dst4_hbm.at[wid, j + 2], dst_c[b], sem_i[b])
                pltpu.async_copy(
                    ex_hbm.at[wid, pl.ds((j + 2) * _CH, _CH)], ex_c[b],
                    sem_i[b])
        return carry

    lax.fori_loop(0, _NCH // 2, pair_body, 0)

    pltpu.make_async_copy(rows1, zn_sh.at[dst_s1.at[0]], sem_s1).wait()

    plsc.subcore_barrier()

    @pl.when(c == 0)
    def _():
        pltpu.sync_copy(zn_sh.at[pl.ds(base, _STR)],
                        out0.at[pl.ds(base, _STR)])

    @pl.when(c == 1)
    def _():
        pltpu.sync_copy(zn_sh.at[pl.ds(base, _STR)],
                        out1.at[pl.ds(base, _STR)])


def _sc_pass2(src4, dst4, ex, dp, z):
    mesh = plsc.VectorSubcoreMesh(core_axis_name="c", subcore_axis_name="s")
    return pl.kernel(
        _sc2_body,
        out_type=[
            jax.ShapeDtypeStruct((_NP, _D), _f32),
            jax.ShapeDtypeStruct((_NP, _D), _f32),
        ],
        mesh=mesh,
        scratch_types=[
            pltpu.VMEM((1, _CH), _i32),
            pltpu.VMEM((1, _CH), _i32),
            pltpu.VMEM((1, _CH), _i32),
            pltpu.VMEM((1, _CH), _i32),
            pltpu.VMEM((_CH,), _f32),
            pltpu.VMEM((_CH,), _f32),
            pltpu.VMEM((1, _CH), _i32),
            pltpu.VMEM((1, _CH), _i32),
            pltpu.VMEM((_CH, _D), _f32),
            pltpu.VMEM((_CH, _D), _f32),
            pltpu.VMEM((_NP,), _f32),
            pltpu.VMEM((_STR,), _f32),
            pltpu.VMEM((_STR,), _f32),
            pltpu.VMEM((_CH,), _f32),
            pltpu.VMEM_SHARED((_NP, _D), _f32),
            pltpu.VMEM_SHARED((_NP,), _f32),
            pltpu.SemaphoreType.DMA,
            pltpu.SemaphoreType.DMA,
            pltpu.SemaphoreType.DMA,
            pltpu.SemaphoreType.DMA,
            pltpu.SemaphoreType.DMA,
            pltpu.SemaphoreType.DMA,
        ],
        compiler_params=pltpu.CompilerParams(needs_layout_passes=False),
    )(src4, dst4, ex, dp, z)


def _epi_body(zi_ref, zn0_ref, zn1_ref, out_ref):
    out_ref[...] = jnp.maximum(
        zi_ref[...] + zn0_ref[...] + zn1_ref[...], 0.0)


def _tc_epi(zi, zn0, zn1):
    nb = 10
    blk = _N // nb
    return pl.pallas_call(
        _epi_body,
        grid=(nb,),
        in_specs=[
            pl.BlockSpec((blk, _D), lambda i: (i, 0)),
            pl.BlockSpec((blk, _D), lambda i: (i, 0)),
            pl.BlockSpec((blk, _D), lambda i: (i, 0)),
        ],
        out_specs=pl.BlockSpec((blk, _D), lambda i: (i, 0)),
        out_shape=jax.ShapeDtypeStruct((_N, _D), _f32),
    )(zi, zn0, zn1)


@jax.jit
def kernel(h, edge_attr, W0, W1, W2, attn_w, weights, edge_index):
    del weights
    src = edge_index[0]
    dst = edge_index[1]

    a1 = attn_w[0, :_D].reshape(_D, 1)
    a2 = attn_w[0, _D:2 * _D].reshape(_D, 1)
    a3 = attn_w[0, 2 * _D:].reshape(16, 1)

    h_p = jnp.pad(h, ((0, _NP - _N), (0, 0)))
    z, zi, s1, s2 = _tc_prep(h_p, W1.T, W2.T, a1, a2)
    u = _tc_u(edge_attr.reshape(_E // 8, _D), W0.T, a3)

    pad_e = _EP - _E
    src_p = jnp.concatenate(
        [src, jnp.zeros((pad_e,), _i32)]).reshape(_NW, _EPT)
    dst_p = jnp.concatenate(
        [dst, jnp.zeros((pad_e,), _i32)]).reshape(_NW, _EPT)
    u_p = jnp.concatenate(
        [u.reshape(_E), jnp.full((pad_e,), -jnp.inf, _f32)]
    ).reshape(_NW, _EPT)

    ex, dp = _sc_pass1(src_p, dst_p, u_p, s1[:, 0], s2[:, 0])
    zn0, zn1 = _sc_pass2(src_p.reshape(_NW, _NCH, 1, _CH),
                         dst_p.reshape(_NW, _NCH, 1, _CH), ex, dp, z)

    return _tc_epi(zi, zn0, zn1)
